# baseline (speedup 1.0000x reference)
import functools

import jax
import jax.numpy as jnp
from jax import lax
from jax.experimental import pallas as pl
from jax.experimental.pallas import tpu as pltpu
from jax.experimental.pallas import tpu_sc as plsc

B = 16384
D = 64
V = 100000
NC = 2
NS = 16
NW = NC * NS
L = 16
TCOLS = (V + 127) // 128
TPW = (TCOLS + NW - 1) // NW
BPW = B // NW
SENT = 0x7FFFFFFF
NPASS_ROWS = 128
GPP = NPASS_ROWS // L

_CP = pltpu.CompilerParams(use_tc_tiling_on_sc=True, needs_layout_passes=False,
                           disable_bounds_checks=True)


def _scan_table(idx_v, wl, lo_tc, n_tc):
    lane = lax.iota(jnp.int32, L)

    def body(t, cnt):
        v = idx_v[pl.ds(t * L, L)]
        tcv = v >> 7
        m = (tcv >= lo_tc) & (tcv < lo_tc + n_tc)
        cs = jnp.cumsum(m.astype(jnp.int32))
        slots = cnt + cs - 1
        w = v * jnp.int32(B) + (t * L + lane)
        plsc.store_scatter(wl, [slots], w, mask=m)
        return cnt + cs[15]

    cnt = lax.fori_loop(0, B // L, body, jnp.int32(0))
    wl[pl.ds(cnt, L)] = jnp.full((L,), SENT, jnp.int32)
    return cnt


def _process_table(table_hbm, out_hbm, wl, cnt, lo_tc,
                   blk, stage, sjl, ml, sem_a, sem_b, sem_s0, sem_s1):
    lane = lax.iota(jnp.int32, L)
    nch = (cnt + L - 1) // L

    def blk_start(k, slot, sem):
        start = pl.multiple_of(jnp.minimum(lo_tc + k, TCOLS - 1) * 128, 128)
        for tr in range(D // 8):
            pltpu.async_copy(
                table_hbm.at[pl.ds(tr * 8, 8), pl.ds(start, 128)],
                slot.at[pl.ds(tr * 8, 8)], sem)

    def blk_drain(slot, sem):
        for tr in range(D // 8):
            pltpu.make_async_copy(
                table_hbm.at[pl.ds(0, 8), pl.ds(0, 128)],
                slot.at[pl.ds(tr * 8, 8)], sem).wait()

    def scat_issue(slot, sem):
        pltpu.async_copy(stage.at[slot], out_hbm.at[sjl.at[slot]], sem)

    def scat_drain(slot, sem):
        pltpu.make_async_copy(
            stage.at[slot], out_hbm.at[sjl.at[slot]], sem).wait()

    def sjl_clear(slot):
        for q in range(GPP):
            sjl[slot, pl.ds(q * L, L)] = jnp.full((L,), B, jnp.int32)

    def process_one(k, blk_slot, slot, sem):
        return
        tc = lo_tc + k
        start = jnp.minimum(tc, TCOLS - 1) * 128

        def f_body(ch, mc):
            w16 = wl[pl.ds(ch * L, L)]
            m = (w16 // jnp.int32(B * 128)) == tc
            cs = jnp.cumsum(m.astype(jnp.int32))
            plsc.store_scatter(ml, [mc + cs - 1], w16, mask=m)
            return mc + cs[15]

        mcount = lax.fori_loop(0, nch, f_body, jnp.int32(0))
        ml[pl.ds(mcount, L)] = jnp.full((L,), SENT, jnp.int32)

        def g_body_at(p):
            def g_body(g, carry):
                wg = ml[pl.ds(p * NPASS_ROWS + g * L, L)]
                fc = jnp.clip(wg // jnp.int32(B) - start, 0, 127)
                jv = jnp.where(wg == SENT, jnp.int32(B), wg % jnp.int32(B))
                rows = g * L + lane
                sjl[slot, pl.ds(g * L, L)] = jv
                for c in range(D):
                    vals = plsc.load_gather(
                        blk_slot, [jnp.full((L,), c, jnp.int32), fc])
                    plsc.store_scatter(
                        stage.at[slot],
                        [rows, jnp.full((L,), c, jnp.int32)], vals)
                return carry
            return g_body

        scat_drain(slot, sem)
        sjl_clear(slot)
        ngrp0 = jnp.minimum(GPP, (mcount + L - 1) // L)
        lax.fori_loop(0, ngrp0, g_body_at(jnp.int32(0)), 0)
        scat_issue(slot, sem)

        npass = (mcount + NPASS_ROWS - 1) // NPASS_ROWS

        @pl.when(npass > 1)
        def _():
            def ov_body(p, carry):
                scat_drain(slot, sem)
                sjl_clear(slot)
                rem = mcount - p * NPASS_ROWS
                ngrp = jnp.minimum(GPP, (rem + L - 1) // L)
                lax.fori_loop(0, ngrp, g_body_at(p), 0)
                scat_issue(slot, sem)
                return carry

            lax.fori_loop(1, npass, ov_body, 0)

    blk_start(0, blk.at[0], sem_a)
    npairs = (TPW + 1) // 2

    def pair_body(kb, carry):
        k0 = 2 * kb
        blk_start(k0 + 1, blk.at[1], sem_b)
        blk_drain(blk.at[0], sem_a)
        process_one(k0, blk.at[0], 0, sem_s0)
        blk_start(k0 + 2, blk.at[0], sem_a)
        blk_drain(blk.at[1], sem_b)
        process_one(k0 + 1, blk.at[1], 1, sem_s1)
        return carry

    lax.fori_loop(0, npairs, pair_body, 0)
    blk_drain(blk.at[0], sem_a)


def _k1_body(guT_hbm, giT_hbm, user_hbm, item_hbm,
             gou_hbm, goi_hbm,
             idxbuf, wl_u, wl_i, blk, stage, sjl,
             sem_idx, sem_a, sem_b, sem_s0, sem_s1):
    wid = lax.axis_index("s") * NC + lax.axis_index("c")
    lo_tc = wid * TPW
    n_tc = jnp.minimum(TPW, jnp.maximum(TCOLS - lo_tc, 1))

    pltpu.async_copy(user_hbm, idxbuf.at[pl.ds(0, B)], sem_idx).wait()
    cnt_u = jnp.int32(0)
    pltpu.async_copy(item_hbm, idxbuf.at[pl.ds(0, B)], sem_idx).wait()
    cnt_i = jnp.int32(0)

    for q in range(GPP):
        sjl[0, pl.ds(q * L, L)] = jnp.full((L,), B, jnp.int32)
        sjl[1, pl.ds(q * L, L)] = jnp.full((L,), B, jnp.int32)
    pltpu.async_copy(stage.at[0], gou_hbm.at[sjl.at[0]], sem_s0)
    pltpu.async_copy(stage.at[1], gou_hbm.at[sjl.at[1]], sem_s1)

    _process_table(guT_hbm, gou_hbm, wl_u, cnt_u, lo_tc,
                   blk, stage, sjl, idxbuf, sem_a, sem_b, sem_s0, sem_s1)
    _process_table(giT_hbm, goi_hbm, wl_i, cnt_i, lo_tc,
                   blk, stage, sjl, idxbuf, sem_a, sem_b, sem_s0, sem_s1)

    pltpu.make_async_copy(
        stage.at[0], goi_hbm.at[sjl.at[0]], sem_s0).wait()
    pltpu.make_async_copy(
        stage.at[1], goi_hbm.at[sjl.at[1]], sem_s1).wait()


K2_SLAB = BPW // 2


def _k2_body(gou_hbm, goi_hbm, xui_hbm, gu_v, gi_v, xui_v, sem_a, sem_b):
    wid = lax.axis_index("s") * NC + lax.axis_index("c")
    base = wid * BPW

    for s in range(2):
        off = base + s * K2_SLAB
        ca = pltpu.async_copy(gou_hbm.at[pl.ds(off, K2_SLAB)], gu_v, sem_a)
        cb = pltpu.async_copy(goi_hbm.at[pl.ds(off, K2_SLAB)], gi_v, sem_b)
        ca.wait()
        cb.wait()

        def group_body(g, carry):
            rows = g * L + lax.iota(jnp.int32, L)
            acc = jnp.zeros((L,), jnp.float32)
            for c in range(D):
                cols = jnp.full((L,), c, jnp.int32)
                u = plsc.load_gather(gu_v, [rows, cols])
                v = plsc.load_gather(gi_v, [rows, cols])
                acc = acc + u * v
            xui_v[pl.ds(s * K2_SLAB + g * L, L)] = acc
            return carry

        lax.fori_loop(0, K2_SLAB // L, group_body, 0)

    pltpu.sync_copy(xui_v, xui_hbm.at[pl.ds(base, BPW)])


@jax.jit
def _run(Gu, Gi, user32, item32):
    mesh = plsc.VectorSubcoreMesh(core_axis_name="c", subcore_axis_name="s")
    k1 = pl.kernel(
        _k1_body,
        out_type=[
            jax.ShapeDtypeStruct((B + 1, 128), jnp.float32),
            jax.ShapeDtypeStruct((B + 1, 128), jnp.float32),
        ],
        mesh=mesh,
        compiler_params=_CP,
        scratch_types=[
            pltpu.VMEM((B + L,), jnp.int32),
            pltpu.VMEM((B + L,), jnp.int32),
            pltpu.VMEM((B + L,), jnp.int32),
            pltpu.VMEM((2, D, 128), jnp.float32),
            pltpu.VMEM((2, NPASS_ROWS, 128), jnp.float32),
            pltpu.VMEM((2, NPASS_ROWS), jnp.int32),
            pltpu.SemaphoreType.DMA,
            pltpu.SemaphoreType.DMA,
            pltpu.SemaphoreType.DMA,
            pltpu.SemaphoreType.DMA,
            pltpu.SemaphoreType.DMA,
        ],
    )
    gou_pad, goi_pad = k1(Gu.T, Gi.T, user32, item32)

    k2 = pl.kernel(
        _k2_body,
        out_type=[jax.ShapeDtypeStruct((B,), jnp.float32)],
        mesh=mesh,
        compiler_params=_CP,
        scratch_types=[
            pltpu.VMEM((K2_SLAB, 128), jnp.float32),
            pltpu.VMEM((K2_SLAB, 128), jnp.float32),
            pltpu.VMEM((BPW,), jnp.float32),
            pltpu.SemaphoreType.DMA,
            pltpu.SemaphoreType.DMA,
        ],
    )
    (xui,) = k2(gou_pad, goi_pad)
    return xui, gou_pad[:B, :D], goi_pad[:B, :D]


def kernel(Gu, Gi, user, item):
    xui, gamma_u, gamma_i = _run(
        Gu, Gi, user.astype(jnp.int32), item.astype(jnp.int32))
    return (xui, gamma_u, gamma_i)


# speedup vs baseline: 2.3092x; 2.3092x over previous
import functools

import jax
import jax.numpy as jnp
from jax import lax
from jax.experimental import pallas as pl
from jax.experimental.pallas import tpu as pltpu
from jax.experimental.pallas import tpu_sc as plsc

B = 16384
D = 64
NC = 2
NS = 16
NW = NC * NS
BPW = B // NW
CH = 128
NCH = BPW // CH

TC_ROWS = 2048


def _sc_body(gu_hbm, gi_hbm, user_hbm, item_hbm,
             gou_hbm, goi_hbm,
             idx_u, idx_i, gu_v, gi_v,
             sem_idx, sem_gat, sem_out):
    wid = lax.axis_index("s") * NC + lax.axis_index("c")
    base = wid * BPW

    cu = pltpu.async_copy(user_hbm.at[wid], idx_u, sem_idx)
    ci = pltpu.async_copy(item_hbm.at[wid], idx_i, sem_idx)
    cu.wait()
    ci.wait()

    gathers = []
    for j in range(NCH):
        gathers.append(pltpu.async_copy(
            gu_hbm.at[idx_u.at[j]], gu_v.at[pl.ds(j * CH, CH)], sem_gat))
        gathers.append(pltpu.async_copy(
            gi_hbm.at[idx_i.at[j]], gi_v.at[pl.ds(j * CH, CH)], sem_gat))
    for c in gathers:
        c.wait()

    ou = pltpu.async_copy(gu_v, gou_hbm.at[pl.ds(base, BPW)], sem_out)
    oi = pltpu.async_copy(gi_v, goi_hbm.at[pl.ds(base, BPW)], sem_out)
    ou.wait()
    oi.wait()


def _tc_body(gu_ref, gi_ref, out_ref):
    out_ref[...] = jnp.sum(gu_ref[...] * gi_ref[...], axis=1)


@jax.jit
def _run(Gu, Gi, user_r, item_r):
    mesh = plsc.VectorSubcoreMesh(core_axis_name="c", subcore_axis_name="s")
    gather_fn = pl.kernel(
        _sc_body,
        out_type=[
            jax.ShapeDtypeStruct((B, D), jnp.float32),
            jax.ShapeDtypeStruct((B, D), jnp.float32),
        ],
        mesh=mesh,
        compiler_params=pltpu.CompilerParams(use_tc_tiling_on_sc=False),
        scratch_types=[
            pltpu.VMEM((NCH, CH), jnp.int32),
            pltpu.VMEM((NCH, CH), jnp.int32),
            pltpu.VMEM((BPW, D), jnp.float32),
            pltpu.VMEM((BPW, D), jnp.float32),
            pltpu.SemaphoreType.DMA,
            pltpu.SemaphoreType.DMA,
            pltpu.SemaphoreType.DMA,
        ],
    )
    gamma_u, gamma_i = gather_fn(Gu, Gi, user_r, item_r)

    xui = pl.pallas_call(
        _tc_body,
        grid=(B // TC_ROWS,),
        in_specs=[
            pl.BlockSpec((TC_ROWS, D), lambda i: (i, 0)),
            pl.BlockSpec((TC_ROWS, D), lambda i: (i, 0)),
        ],
        out_specs=pl.BlockSpec((TC_ROWS,), lambda i: (i,)),
        out_shape=jax.ShapeDtypeStruct((B,), jnp.float32),
    )(gamma_u, gamma_i)

    return xui, gamma_u, gamma_i


def kernel(Gu, Gi, user, item):
    user_r = user.astype(jnp.int32).reshape(NW, NCH, CH)
    item_r = item.astype(jnp.int32).reshape(NW, NCH, CH)
    xui, gamma_u, gamma_i = _run(Gu, Gi, user_r, item_r)
    return (xui, gamma_u, gamma_i)


# speedup vs baseline: 2.4504x; 1.0611x over previous
import functools

import jax
import jax.numpy as jnp
from jax import lax
from jax.experimental import pallas as pl
from jax.experimental.pallas import tpu as pltpu
from jax.experimental.pallas import tpu_sc as plsc

B = 16384
D = 64
NC = 2
NS = 16
NW = NC * NS
BPW = B // NW
CH = 128
NCH = BPW // CH

TC_ROWS = 2048


def _sc_body(gu_hbm, gi_hbm, user_hbm, item_hbm,
             gou_hbm, goi_hbm,
             idx_u, idx_i, gu_v, gi_v,
             sem_idx, sem_gat, sem_out):
    wid = lax.axis_index("s") * NC + lax.axis_index("c")
    base = wid * BPW

    cu = pltpu.async_copy(user_hbm.at[wid], idx_u, sem_idx)
    ci = pltpu.async_copy(item_hbm.at[wid], idx_i, sem_idx)
    cu.wait()
    ci.wait()

    gathers = []
    for j in range(NCH):
        gathers.append(pltpu.async_copy(
            gu_hbm.at[idx_u.at[j]], gu_v.at[pl.ds(j * CH, CH)], sem_gat))
        gathers.append(pltpu.async_copy(
            gi_hbm.at[idx_i.at[j]], gi_v.at[pl.ds(j * CH, CH)], sem_gat))
    for c in gathers:
        c.wait()

    ou = pltpu.async_copy(gu_v, gou_hbm.at[pl.ds(base, BPW)], sem_out)
    oi = pltpu.async_copy(gi_v, goi_hbm.at[pl.ds(base, BPW)], sem_out)
    ou.wait()
    oi.wait()


def _tc_body(gu_ref, gi_ref, xui_ref, guT_ref, giT_ref):
    gu = gu_ref[...]
    gi = gi_ref[...]
    xui_ref[...] = jnp.sum(gu * gi, axis=1)
    guT_ref[...] = gu.T
    giT_ref[...] = gi.T


@jax.jit
def _run(Gu, Gi, user_r, item_r):
    mesh = plsc.VectorSubcoreMesh(core_axis_name="c", subcore_axis_name="s")
    gather_fn = pl.kernel(
        _sc_body,
        out_type=[
            jax.ShapeDtypeStruct((B, D), jnp.float32),
            jax.ShapeDtypeStruct((B, D), jnp.float32),
        ],
        mesh=mesh,
        compiler_params=pltpu.CompilerParams(use_tc_tiling_on_sc=False),
        scratch_types=[
            pltpu.VMEM((NCH, CH), jnp.int32),
            pltpu.VMEM((NCH, CH), jnp.int32),
            pltpu.VMEM((BPW, D), jnp.float32),
            pltpu.VMEM((BPW, D), jnp.float32),
            pltpu.SemaphoreType.DMA,
            pltpu.SemaphoreType.DMA,
            pltpu.SemaphoreType.DMA,
        ],
    )
    gamma_u, gamma_i = gather_fn(Gu, Gi, user_r, item_r)

    xui, guT, giT = pl.pallas_call(
        _tc_body,
        grid=(B // TC_ROWS,),
        in_specs=[
            pl.BlockSpec((TC_ROWS, D), lambda i: (i, 0)),
            pl.BlockSpec((TC_ROWS, D), lambda i: (i, 0)),
        ],
        out_specs=[
            pl.BlockSpec((TC_ROWS,), lambda i: (i,)),
            pl.BlockSpec((D, TC_ROWS), lambda i: (0, i)),
            pl.BlockSpec((D, TC_ROWS), lambda i: (0, i)),
        ],
        out_shape=[
            jax.ShapeDtypeStruct((B,), jnp.float32),
            jax.ShapeDtypeStruct((D, B), jnp.float32),
            jax.ShapeDtypeStruct((D, B), jnp.float32),
        ],
    )(gamma_u, gamma_i)

    return xui, guT.T, giT.T


def kernel(Gu, Gi, user, item):
    user_r = user.astype(jnp.int32).reshape(NW, NCH, CH)
    item_r = item.astype(jnp.int32).reshape(NW, NCH, CH)
    xui, gamma_u, gamma_i = _run(Gu, Gi, user_r, item_r)
    return (xui, gamma_u, gamma_i)


# speedup vs baseline: 2.6787x; 1.0932x over previous
import functools

import jax
import jax.numpy as jnp
from jax import lax
from jax.experimental import pallas as pl
from jax.experimental.pallas import tpu as pltpu
from jax.experimental.pallas import tpu_sc as plsc

B = 16384
D = 64
NC = 2
NS = 16
NW = NC * NS
BPW = B // NW
CH = 128
NCH = BPW // CH

TC_ROWS = 2048


def _sc_body(gu_hbm, gi_hbm, user_hbm, item_hbm,
             gou_hbm, goi_hbm,
             idx_u, idx_i, gu_v, gi_v,
             sem_idx, sem_gat, sem_out):
    wid = lax.axis_index("s") * NC + lax.axis_index("c")
    base = wid * BPW

    cu = pltpu.async_copy(user_hbm.at[wid], idx_u, sem_idx)
    ci = pltpu.async_copy(item_hbm.at[wid], idx_i, sem_idx)
    cu.wait()
    ci.wait()

    gathers = []
    for j in range(NCH):
        gathers.append(pltpu.async_copy(
            gu_hbm.at[idx_u.at[j]], gu_v.at[pl.ds(j * CH, CH)], sem_gat))
        gathers.append(pltpu.async_copy(
            gi_hbm.at[idx_i.at[j]], gi_v.at[pl.ds(j * CH, CH)], sem_gat))
    for c in gathers:
        c.wait()

    ou = pltpu.async_copy(
        gu_v, gou_hbm.at[pl.ds(base, BPW), pl.ds(0, D)], sem_out)
    oi = pltpu.async_copy(
        gi_v, goi_hbm.at[pl.ds(base, BPW), pl.ds(0, D)], sem_out)
    ou.wait()
    oi.wait()


def _tc_body(gu_ref, gi_ref, xui_ref, guT_ref, giT_ref):
    gu = gu_ref[:, :D]
    gi = gi_ref[:, :D]
    xui_ref[...] = jnp.sum(gu * gi, axis=1)
    guT_ref[...] = gu.T
    giT_ref[...] = gi.T


@jax.jit
def _run(Gu, Gi, user_r, item_r):
    mesh = plsc.VectorSubcoreMesh(core_axis_name="c", subcore_axis_name="s")
    gather_fn = pl.kernel(
        _sc_body,
        out_type=[
            jax.ShapeDtypeStruct((B, 128), jnp.float32),
            jax.ShapeDtypeStruct((B, 128), jnp.float32),
        ],
        mesh=mesh,
        compiler_params=pltpu.CompilerParams(use_tc_tiling_on_sc=False),
        scratch_types=[
            pltpu.VMEM((NCH, CH), jnp.int32),
            pltpu.VMEM((NCH, CH), jnp.int32),
            pltpu.VMEM((BPW, D), jnp.float32),
            pltpu.VMEM((BPW, D), jnp.float32),
            pltpu.SemaphoreType.DMA,
            pltpu.SemaphoreType.DMA,
            pltpu.SemaphoreType.DMA,
        ],
    )
    gamma_u, gamma_i = gather_fn(Gu, Gi, user_r, item_r)

    xui, guT, giT = pl.pallas_call(
        _tc_body,
        grid=(B // TC_ROWS,),
        in_specs=[
            pl.BlockSpec((TC_ROWS, 128), lambda i: (i, 0)),
            pl.BlockSpec((TC_ROWS, 128), lambda i: (i, 0)),
        ],
        out_specs=[
            pl.BlockSpec((TC_ROWS,), lambda i: (i,)),
            pl.BlockSpec((D, TC_ROWS), lambda i: (0, i)),
            pl.BlockSpec((D, TC_ROWS), lambda i: (0, i)),
        ],
        out_shape=[
            jax.ShapeDtypeStruct((B,), jnp.float32),
            jax.ShapeDtypeStruct((D, B), jnp.float32),
            jax.ShapeDtypeStruct((D, B), jnp.float32),
        ],
    )(gamma_u, gamma_i)

    return xui, guT.T, giT.T


def kernel(Gu, Gi, user, item):
    user_r = user.astype(jnp.int32).reshape(NW, NCH, CH)
    item_r = item.astype(jnp.int32).reshape(NW, NCH, CH)
    xui, gamma_u, gamma_i = _run(Gu, Gi, user_r, item_r)
    return (xui, gamma_u, gamma_i)


# speedup vs baseline: 2.7095x; 1.0115x over previous
import functools

import jax
import jax.numpy as jnp
from jax import lax
from jax.experimental import pallas as pl
from jax.experimental.pallas import tpu as pltpu
from jax.experimental.pallas import tpu_sc as plsc

B = 16384
D = 64
NC = 2
NS = 16
NW = NC * NS
BPW = B // NW
CH = 128
NCH = BPW // CH

TC_ROWS = 2048


def _sc_body(tab_hbm, idx_hbm, out_hbm, idx_v, rows_v, sem_idx, sem_gat,
             sem_out):
    wid = lax.axis_index("s") * NC + lax.axis_index("c")
    base = wid * BPW

    pltpu.async_copy(idx_hbm.at[wid], idx_v, sem_idx).wait()

    gathers = []
    for j in range(NCH):
        gathers.append(pltpu.async_copy(
            tab_hbm.at[idx_v.at[j]], rows_v.at[pl.ds(j * CH, CH)], sem_gat))
    for c in gathers:
        c.wait()

    pltpu.async_copy(
        rows_v, out_hbm.at[pl.ds(base, BPW), pl.ds(0, D)], sem_out).wait()


def _tc_body(gu_ref, gi_ref, xui_ref, guT_ref, giT_ref):
    gu = gu_ref[:, :D]
    gi = gi_ref[:, :D]
    xui_ref[...] = jnp.sum(gu * gi, axis=1)
    guT_ref[...] = gu.T
    giT_ref[...] = gi.T


@jax.jit
def _run(Gu, Gi, user_r, item_r):
    mesh = plsc.VectorSubcoreMesh(core_axis_name="c", subcore_axis_name="s")
    gather_fn = pl.kernel(
        _sc_body,
        out_type=[jax.ShapeDtypeStruct((B, 128), jnp.float32)],
        mesh=mesh,
        compiler_params=pltpu.CompilerParams(use_tc_tiling_on_sc=False),
        scratch_types=[
            pltpu.VMEM((NCH, CH), jnp.int32),
            pltpu.VMEM((BPW, D), jnp.float32),
            pltpu.SemaphoreType.DMA,
            pltpu.SemaphoreType.DMA,
            pltpu.SemaphoreType.DMA,
        ],
    )
    (gamma_u,) = gather_fn(Gu, user_r)
    (gamma_i,) = gather_fn(Gi, item_r)

    xui, guT, giT = pl.pallas_call(
        _tc_body,
        grid=(B // TC_ROWS,),
        in_specs=[
            pl.BlockSpec((TC_ROWS, 128), lambda i: (i, 0)),
            pl.BlockSpec((TC_ROWS, 128), lambda i: (i, 0)),
        ],
        out_specs=[
            pl.BlockSpec((TC_ROWS,), lambda i: (i,)),
            pl.BlockSpec((D, TC_ROWS), lambda i: (0, i)),
            pl.BlockSpec((D, TC_ROWS), lambda i: (0, i)),
        ],
        out_shape=[
            jax.ShapeDtypeStruct((B,), jnp.float32),
            jax.ShapeDtypeStruct((D, B), jnp.float32),
            jax.ShapeDtypeStruct((D, B), jnp.float32),
        ],
    )(gamma_u, gamma_i)

    return xui, guT.T, giT.T


def kernel(Gu, Gi, user, item):
    user_r = user.astype(jnp.int32).reshape(NW, NCH, CH)
    item_r = item.astype(jnp.int32).reshape(NW, NCH, CH)
    xui, gamma_u, gamma_i = _run(Gu, Gi, user_r, item_r)
    return (xui, gamma_u, gamma_i)
